# baseline (speedup 1.0000x reference)
import functools

import jax
import jax.numpy as jnp
from jax.experimental import pallas as pl
from jax.experimental.pallas import tpu as pltpu

E = 8
K = 2
D = 768
T = 8192
BT = 512


def _moe_body(x_ref, wgt_ref, wet_ref, be_ref, out_ref):
    x = x_ref[...]
    logits = jnp.dot(x, wgt_ref[...], preferred_element_type=jnp.float32)
    iota = jax.lax.broadcasted_iota(jnp.int32, (BT, E), 1)
    v1 = jnp.max(logits, axis=1, keepdims=True)
    i1 = jnp.min(jnp.where(logits == v1, iota, E), axis=1, keepdims=True)
    oh1 = iota == i1
    masked = jnp.where(oh1, -jnp.inf, logits)
    v2 = jnp.max(masked, axis=1, keepdims=True)
    i2 = jnp.min(jnp.where(masked == v2, iota, E), axis=1, keepdims=True)
    oh2 = iota == i2
    t = jnp.exp(v2 - v1)
    denom = 1.0 + t
    w = jnp.where(oh1, 1.0 / denom, 0.0) + jnp.where(oh2, t / denom, 0.0)

    xb = x.astype(jnp.bfloat16)
    wbias = jnp.dot(w, be_ref[...], preferred_element_type=jnp.float32)
    terms = []
    for e in range(E):
        ye = jnp.dot(
            xb, wet_ref[:, D * e : D * (e + 1)], preferred_element_type=jnp.float32
        )
        terms.append(w[:, e : e + 1] * ye)
    s01 = terms[0] + terms[1]
    s23 = terms[2] + terms[3]
    s45 = terms[4] + terms[5]
    s67 = terms[6] + terms[7]
    out_ref[...] = ((s01 + s23) + (s45 + s67)) + wbias


@jax.jit
def _moe(inputs, wgt, wet, be):
    grid = T // BT
    return pl.pallas_call(
        _moe_body,
        grid=(grid,),
        in_specs=[
            pl.BlockSpec((BT, D), lambda i: (i, 0)),
            pl.BlockSpec((D, E), lambda i: (0, 0)),
            pl.BlockSpec((D, E * D), lambda i: (0, 0)),
            pl.BlockSpec((E, D), lambda i: (0, 0)),
        ],
        out_specs=pl.BlockSpec((BT, D), lambda i: (i, 0)),
        out_shape=jax.ShapeDtypeStruct((T, D), jnp.float32),
    )(inputs, wgt, wet, be)


def kernel(inputs, Wg, We, be):
    wgt = Wg.T
    wet = (
        jnp.swapaxes(We, 1, 2).astype(jnp.bfloat16)
        .transpose(1, 0, 2).reshape(D, E * D)
    )
    return _moe(inputs, wgt, wet, be)


# speedup vs baseline: 1.0589x; 1.0589x over previous
import functools

import jax
import jax.numpy as jnp
from jax.experimental import pallas as pl
from jax.experimental.pallas import tpu as pltpu

E = 8
K = 2
D = 768
T = 8192
BT = 512


def _moe_body(x_ref, wgt_ref, wet_ref, be_ref, out_ref):
    xb = x_ref[...]
    logits = jnp.dot(xb, wgt_ref[...], preferred_element_type=jnp.float32)
    iota = jax.lax.broadcasted_iota(jnp.int32, (BT, E), 1)
    v1 = jnp.max(logits, axis=1, keepdims=True)
    i1 = jnp.min(jnp.where(logits == v1, iota, E), axis=1, keepdims=True)
    oh1 = iota == i1
    masked = jnp.where(oh1, -jnp.inf, logits)
    v2 = jnp.max(masked, axis=1, keepdims=True)
    i2 = jnp.min(jnp.where(masked == v2, iota, E), axis=1, keepdims=True)
    oh2 = iota == i2
    t = jnp.exp(v2 - v1)
    denom = 1.0 + t
    w = jnp.where(oh1, 1.0 / denom, 0.0) + jnp.where(oh2, t / denom, 0.0)

    acc0 = jnp.zeros((BT, D), dtype=jnp.float32)
    acc1 = jnp.zeros((BT, D), dtype=jnp.float32)
    for e in range(E):
        ye = jnp.dot(xb, wet_ref[e], preferred_element_type=jnp.float32)
        t = w[:, e : e + 1] * (ye + be_ref[e][None, :])
        if e % 2 == 0:
            acc0 = acc0 + t
        else:
            acc1 = acc1 + t
    out_ref[...] = acc0 + acc1


@jax.jit
def _moe(inputs, wgt, wet, be):
    grid = T // BT
    return pl.pallas_call(
        _moe_body,
        grid=(grid,),
        in_specs=[
            pl.BlockSpec((BT, D), lambda i: (i, 0)),
            pl.BlockSpec((D, E), lambda i: (0, 0)),
            pl.BlockSpec((E, D, D), lambda i: (0, 0, 0)),
            pl.BlockSpec((E, D), lambda i: (0, 0)),
        ],
        out_specs=pl.BlockSpec((BT, D), lambda i: (i, 0)),
        out_shape=jax.ShapeDtypeStruct((T, D), jnp.float32),
    )(inputs, wgt, wet, be)


def kernel(inputs, Wg, We, be):
    xb = inputs.astype(jnp.bfloat16)
    wgt = Wg.T.astype(jnp.bfloat16)
    wet = jnp.swapaxes(We, 1, 2).astype(jnp.bfloat16)
    return _moe(xb, wgt, wet, be)


# speedup vs baseline: 1.3688x; 1.2927x over previous
import jax
import jax.numpy as jnp
from jax.experimental import pallas as pl

E = 8
K = 2
D = 768
T = 8192
BT = 512

_DN = (((1,), (1,)), ((), ()))


def _moe_body(x_ref, wg_ref, we_ref, be_ref, out_ref):
    x = x_ref[...]
    logits = jax.lax.dot_general(
        x, wg_ref[...], _DN, preferred_element_type=jnp.float32
    )
    iota = jax.lax.broadcasted_iota(jnp.int32, (BT, E), 1)
    v1 = jnp.max(logits, axis=1, keepdims=True)
    i1 = jnp.min(jnp.where(logits == v1, iota, E), axis=1, keepdims=True)
    oh1 = iota == i1
    masked = jnp.where(oh1, -jnp.inf, logits)
    v2 = jnp.max(masked, axis=1, keepdims=True)
    i2 = jnp.min(jnp.where(masked == v2, iota, E), axis=1, keepdims=True)
    oh2 = iota == i2
    t = jnp.exp(v2 - v1)
    denom = 1.0 + t
    w = jnp.where(oh1, 1.0 / denom, 0.0) + jnp.where(oh2, t / denom, 0.0)

    acc = jnp.zeros((BT, D), dtype=jnp.float32)
    for e in range(E):
        y = jax.lax.dot_general(
            x, we_ref[e], _DN, preferred_element_type=jnp.float32
        )
        acc = acc + w[:, e : e + 1] * (y + be_ref[e][None, :])
    out_ref[...] = acc


@jax.jit
def _moe(inputs, wg, we, be):
    grid = T // BT
    return pl.pallas_call(
        _moe_body,
        grid=(grid,),
        in_specs=[
            pl.BlockSpec((BT, D), lambda i: (i, 0)),
            pl.BlockSpec((E, D), lambda i: (0, 0)),
            pl.BlockSpec((E, D, D), lambda i: (0, 0, 0)),
            pl.BlockSpec((E, D), lambda i: (0, 0)),
        ],
        out_specs=pl.BlockSpec((BT, D), lambda i: (i, 0)),
        out_shape=jax.ShapeDtypeStruct((T, D), jnp.float32),
    )(inputs, wg, we, be)


def kernel(inputs, Wg, We, be):
    return _moe(inputs, Wg, We, be)


# speedup vs baseline: 1.4254x; 1.0413x over previous
import jax
import jax.numpy as jnp
from jax.experimental import pallas as pl

E = 8
K = 2
D = 768
T = 8192
BT = 1024

_DN = (((1,), (1,)), ((), ()))


def _moe_body(x_ref, wg_ref, we_ref, be_ref, out_ref):
    x = x_ref[...]
    logits = jax.lax.dot_general(
        x, wg_ref[...], _DN, preferred_element_type=jnp.float32
    )
    iota = jax.lax.broadcasted_iota(jnp.int32, (BT, E), 1)
    v1 = jnp.max(logits, axis=1, keepdims=True)
    i1 = jnp.min(jnp.where(logits == v1, iota, E), axis=1, keepdims=True)
    oh1 = iota == i1
    masked = jnp.where(oh1, -jnp.inf, logits)
    v2 = jnp.max(masked, axis=1, keepdims=True)
    i2 = jnp.min(jnp.where(masked == v2, iota, E), axis=1, keepdims=True)
    oh2 = iota == i2
    t = jnp.exp(v2 - v1)
    denom = 1.0 + t
    w = jnp.where(oh1, 1.0 / denom, 0.0) + jnp.where(oh2, t / denom, 0.0)

    acc = jnp.zeros((BT, D), dtype=jnp.float32)
    for e in range(E):
        y = jax.lax.dot_general(
            x, we_ref[e], _DN, preferred_element_type=jnp.float32
        )
        acc = acc + w[:, e : e + 1] * (y + be_ref[e][None, :])
    out_ref[...] = acc


@jax.jit
def _moe(inputs, wg, we, be):
    grid = T // BT
    return pl.pallas_call(
        _moe_body,
        grid=(grid,),
        in_specs=[
            pl.BlockSpec((BT, D), lambda i: (i, 0)),
            pl.BlockSpec((E, D), lambda i: (0, 0)),
            pl.BlockSpec((E, D, D), lambda i: (0, 0, 0)),
            pl.BlockSpec((E, D), lambda i: (0, 0)),
        ],
        out_specs=pl.BlockSpec((BT, D), lambda i: (i, 0)),
        out_shape=jax.ShapeDtypeStruct((T, D), jnp.float32),
    )(inputs, wg, we, be)


def kernel(inputs, Wg, We, be):
    return _moe(inputs, Wg, We, be)
